# Initial kernel scaffold; baseline (speedup 1.0000x reference)
import jax
import jax.numpy as jnp
from jax.experimental import pallas as pl


def kernel(x, edge_index, edge_weight, W0, b0, W1, b1):
    raise NotImplementedError("write your pallas kernel here")


# speedup vs baseline: 6.2556x; 6.2556x over previous
import functools

import jax
import jax.numpy as jnp
from jax import lax
from jax.experimental import pallas as pl
from jax.experimental.pallas import tpu as pltpu
from jax.experimental.pallas import tpu_sc as plsc

N = 10000
D = 128
E = 320000
NP = 10240
NW = 32
EPW = E // NW
KC = 80
KD = 400
BR = 640
NROW = NP // 128


def _zero_rows(ref, nrows):
    def body(i, _):
        for j in range(ref.shape[-1] // 16):
            ref[i, pl.ds(j * 16, 16)] = jnp.zeros((16,), jnp.float32)
        return 0

    lax.fori_loop(0, nrows, body, 0)


def _deg_body(row_hbm, col_hbm, out_hbm, zbuf, rowv, colv, onesv, degs):
    cid = lax.axis_index("c")
    sid = lax.axis_index("s")
    wid = cid * 16 + sid
    nslice = NP // 16

    def zbody(i, _):
        zbuf[pl.ds(i * 16, 16)] = jnp.zeros((16,), jnp.float32)
        return 0

    lax.fori_loop(0, nslice // 16, zbody, 0)
    pltpu.sync_copy(zbuf, degs.at[pl.ds(sid * nslice, nslice)])
    plsc.subcore_barrier()

    one = jnp.full((16,), 1.0, jnp.float32)
    zero = jnp.zeros((16,), jnp.float32)

    def chunk(ch, _):
        base = wid * EPW + ch * KC
        pltpu.sync_copy(row_hbm.at[pl.ds(base, KC)], rowv)
        pltpu.sync_copy(col_hbm.at[pl.ds(base, KC)], colv)

        def group(g, _):
            r = rowv[pl.ds(g * 16, 16)]
            c = colv[pl.ds(g * 16, 16)]
            onesv[pl.ds(g * 16, 16)] = jnp.where(r != c, one, zero)
            return 0

        lax.fori_loop(0, KC // 16, group, 0)
        pltpu.sync_copy(onesv, degs.at[colv], add=True)
        return 0

    lax.fori_loop(0, EPW // KC, chunk, 0)
    plsc.subcore_barrier()
    pltpu.sync_copy(degs.at[pl.ds(sid * nslice, nslice)],
                    out_hbm.at[cid, pl.ds(sid * nslice, nslice)])


def _deg_partials(row, col):
    mesh = plsc.VectorSubcoreMesh(core_axis_name="c", subcore_axis_name="s")
    k = pl.kernel(
        _deg_body,
        out_type=jax.ShapeDtypeStruct((2, NP), jnp.float32),
        mesh=mesh,
        scratch_types=[
            pltpu.VMEM((NP // 16,), jnp.float32),
            pltpu.VMEM((KC,), jnp.int32),
            pltpu.VMEM((KC,), jnp.int32),
            pltpu.VMEM((KC,), jnp.float32),
            pltpu.VMEM_SHARED((NP,), jnp.float32),
        ],
    )
    return k(row, col)


def _msg_body(row_hbm, col_hbm, ew_hbm, yp_hbm, out_hbm, rowv, colv, ewv, buf,
              acc, sem):
    cid = lax.axis_index("c")
    sid = lax.axis_index("s")
    wid = cid * 16 + sid

    _zero_rows(buf, KC)
    for t in range(NP // 16 // KC):
        pltpu.sync_copy(buf, acc.at[pl.ds(sid * (NP // 16) + t * KC, KC)])
    plsc.subcore_barrier()

    def chunk(ch, _):
        base = wid * EPW + ch * KC
        pltpu.sync_copy(row_hbm.at[pl.ds(base, KC)], rowv)
        pltpu.sync_copy(col_hbm.at[pl.ds(base, KC)], colv)
        pltpu.sync_copy(ew_hbm.at[pl.ds(base, KC)], ewv)
        pltpu.async_copy(yp_hbm.at[rowv], buf, sem).wait()

        def scale(g, _):
            w = ewv[pl.ds(g * 16, 16)]
            for lane in range(16):
                s = w[lane]
                e = g * 16 + lane
                for j in range(D // 16):
                    buf[e, pl.ds(j * 16, 16)] = buf[e, pl.ds(j * 16, 16)] * s
            return 0

        lax.fori_loop(0, KC // 16, scale, 0)
        pltpu.sync_copy(buf, acc.at[colv], add=True)
        return 0

    lax.fori_loop(0, EPW // KC, chunk, 0)
    plsc.subcore_barrier()

    for t in range(NP // 16 // KC):
        off = sid * (NP // 16) + t * KC
        pltpu.sync_copy(acc.at[pl.ds(off, KC)], out_hbm.at[cid, pl.ds(off, KC)])


def _msg_partials(row, col, ew, yp):
    mesh = plsc.VectorSubcoreMesh(core_axis_name="c", subcore_axis_name="s")
    k = pl.kernel(
        _msg_body,
        out_type=jax.ShapeDtypeStruct((2, NP, D), jnp.float32),
        mesh=mesh,
        scratch_types=[
            pltpu.VMEM((KC,), jnp.int32),
            pltpu.VMEM((KC,), jnp.int32),
            pltpu.VMEM((KC,), jnp.float32),
            pltpu.VMEM((KC, D), jnp.float32),
            pltpu.VMEM_SHARED((NP, D), jnp.float32),
            pltpu.SemaphoreType.DMA,
        ],
    )
    return k(row, col, ew, yp)


def _first_body(degp_ref, x_ref, w_ref, b_ref, yp_ref, dis_ref):
    deg = jnp.sum(degp_ref[...], axis=0) + 1.0
    dis = lax.rsqrt(deg)[:, None]
    y = jnp.dot(x_ref[...], w_ref[...], preferred_element_type=jnp.float32)
    yp_ref[...] = (y + b_ref[...]) * dis
    dis_ref[...] = dis


def _first_stage(degp, x, wt, b):
    return pl.pallas_call(
        _first_body,
        grid=(NP // BR,),
        in_specs=[
            pl.BlockSpec((2, BR), lambda i: (0, i)),
            pl.BlockSpec((BR, D), lambda i: (i, 0)),
            pl.BlockSpec((D, D), lambda i: (0, 0)),
            pl.BlockSpec((1, D), lambda i: (0, 0)),
        ],
        out_specs=[
            pl.BlockSpec((BR, D), lambda i: (i, 0)),
            pl.BlockSpec((BR, 1), lambda i: (i, 0)),
        ],
        out_shape=[
            jax.ShapeDtypeStruct((NP, D), jnp.float32),
            jax.ShapeDtypeStruct((NP, 1), jnp.float32),
        ],
    )(degp, x, wt, b)


def _mid_body(sp_ref, yp_ref, lw_ref, dis_ref, w_ref, b_ref, out_ref):
    s = sp_ref[0] + sp_ref[1] + lw_ref[...] * yp_ref[...]
    h = jnp.maximum(s * dis_ref[...], 0.0)
    y = jnp.dot(h, w_ref[...], preferred_element_type=jnp.float32)
    out_ref[...] = (y + b_ref[...]) * dis_ref[...]


def _mid_stage(sp, yp, lw, dis, wt, b):
    return pl.pallas_call(
        _mid_body,
        grid=(NP // BR,),
        in_specs=[
            pl.BlockSpec((2, BR, D), lambda i: (0, i, 0)),
            pl.BlockSpec((BR, D), lambda i: (i, 0)),
            pl.BlockSpec((BR, 1), lambda i: (i, 0)),
            pl.BlockSpec((BR, 1), lambda i: (i, 0)),
            pl.BlockSpec((D, D), lambda i: (0, 0)),
            pl.BlockSpec((1, D), lambda i: (0, 0)),
        ],
        out_specs=pl.BlockSpec((BR, D), lambda i: (i, 0)),
        out_shape=jax.ShapeDtypeStruct((NP, D), jnp.float32),
    )(sp, yp, lw, dis, wt, b)


def _final_body(sp_ref, yp_ref, lw_ref, dis_ref, out_ref):
    s = sp_ref[0] + sp_ref[1] + lw_ref[...] * yp_ref[...]
    out_ref[...] = jax.nn.sigmoid(s * dis_ref[...])


def _final_stage(sp, yp, lw, dis):
    return pl.pallas_call(
        _final_body,
        grid=(NP // BR,),
        in_specs=[
            pl.BlockSpec((2, BR, D), lambda i: (0, i, 0)),
            pl.BlockSpec((BR, D), lambda i: (i, 0)),
            pl.BlockSpec((BR, 1), lambda i: (i, 0)),
            pl.BlockSpec((BR, 1), lambda i: (i, 0)),
        ],
        out_specs=pl.BlockSpec((BR, D), lambda i: (i, 0)),
        out_shape=jax.ShapeDtypeStruct((NP, D), jnp.float32),
    )(sp, yp, lw, dis)


@jax.jit
def kernel(x, edge_index, edge_weight, W0, b0, W1, b1):
    row = edge_index[0]
    col = edge_index[1]
    mask = row != col
    ew = jnp.where(mask, edge_weight[:, 0], 0.0)

    scatter_idx = jnp.where(mask, N, row)
    loopw = jnp.ones((N + 1,), jnp.float32).at[scatter_idx].set(
        edge_weight[:, 0])[:N]
    lw = jnp.concatenate([loopw, jnp.ones((NP - N,), jnp.float32)])[:, None]

    xp = jnp.zeros((NP, D), jnp.float32).at[:N].set(x)

    degp = _deg_partials(row, col)
    yp0, dis = _first_stage(degp, xp, W0.T, b0[None, :])
    sp0 = _msg_partials(row, col, ew, yp0)
    yp1 = _mid_stage(sp0, yp0, lw, dis, W1.T, b1[None, :])
    sp1 = _msg_partials(row, col, ew, yp1)
    out = _final_stage(sp1, yp1, lw, dis)
    return out[:N]


# speedup vs baseline: 11.2269x; 1.7947x over previous
import jax
import jax.numpy as jnp
from jax import lax
from jax.experimental import pallas as pl
from jax.experimental.pallas import tpu as pltpu
from jax.experimental.pallas import tpu_sc as plsc

N = 10000
D = 128
E = 320000
NP = 10240
NA = NP + 8
NW = 32
EPW = E // NW
KC = 128
NCH = 80
NSB = 5
BR = 640
NSL = NP // 16
QS = 1.0 / 8388608.0


def _chunk_rows(ch):
    return 3 * lax.bitwise_and(ch, 31)


def _sb_load(edges_hbm, sblk, wid, f):
    half = lax.bitwise_and(f, 1)
    pltpu.sync_copy(edges_hbm.at[wid, f], sblk.at[pl.ds(half * 48, 48)])


def _deg_body(edges_hbm, deg_hbm, lw_hbm, zbuf, sblk, ones0, ones1, lidx0,
              lidx1, ewf0, ewf1, degs, loops, semA0, semA1, semB0, semB1):
    cid = lax.axis_index("c")
    sid = lax.axis_index("s")
    wid = cid * 16 + sid

    def zbody(i, _):
        zbuf[pl.ds(i * 16, 16)] = jnp.zeros((16,), jnp.float32)
        return 0

    lax.fori_loop(0, NSL // 16, zbody, 0)
    pltpu.sync_copy(zbuf, degs.at[pl.ds(sid * NSL, NSL)])

    def obody(i, _):
        zbuf[pl.ds(i * 16, 16)] = jnp.full((16,), 1.0, jnp.float32)
        return 0

    lax.fori_loop(0, NSL // 16, obody, 0)
    pltpu.sync_copy(zbuf, loops.at[pl.ds(sid * NSL, NSL)])

    @pl.when(sid == 0)
    def _():
        pltpu.sync_copy(zbuf.at[pl.ds(0, 8)], degs.at[pl.ds(NP, 8)])
        pltpu.sync_copy(zbuf.at[pl.ds(0, 8)], loops.at[pl.ds(NP, 8)])

    plsc.subcore_barrier()

    one = jnp.full((16,), 1.0, jnp.float32)
    zero = jnp.zeros((16,), jnp.float32)
    trash = jnp.full((16,), NP, jnp.int32)

    _sb_load(edges_hbm, sblk, wid, 0)

    def compute(ch, onesv, lidxv, ewf):
        base = _chunk_rows(ch)

        def group(g, cntv):
            r = sblk[base, pl.ds(g * 16, 16)]
            c = sblk[base + 1, pl.ds(g * 16, 16)]
            q = sblk[base + 2, pl.ds(g * 16, 16)]
            is_loop = r == c
            ones = jnp.where(is_loop, zero, one)
            onesv[pl.ds(g * 16, 16)] = ones
            lidxv[pl.ds(g * 16, 16)] = jnp.where(is_loop, c, trash)
            ewf[pl.ds(g * 16, 16)] = q.astype(jnp.float32) * QS
            return cntv + (one - ones)

        cntv = lax.fori_loop(0, KC // 16, group, zero)
        cnt = cntv[0]
        for k in range(1, 16):
            cnt = cnt + cntv[k]
        return cnt

    def step(i, _):
        @pl.when(jnp.logical_and(lax.bitwise_and(i, 7) == 0,
                                 lax.shift_right_logical(i, 3) + 1 < NSB))
        def _():
            _sb_load(edges_hbm, sblk, wid, lax.shift_right_logical(i, 3) + 1)

        a = 2 * i
        b = 2 * i + 1
        cnt0 = compute(a, ones0, lidx0, ewf0)
        pltpu.async_copy(ones0, degs.at[sblk.at[_chunk_rows(a) + 1]], semA0,
                         add=True)

        @pl.when(cnt0 > 0.0)
        def _():
            pltpu.async_copy(ewf0, loops.at[lidx0], semB0)

        cnt1 = compute(b, ones1, lidx1, ewf1)
        pltpu.async_copy(ones1, degs.at[sblk.at[_chunk_rows(b) + 1]], semA1,
                         add=True)

        @pl.when(cnt1 > 0.0)
        def _():
            pltpu.async_copy(ewf1, loops.at[lidx1], semB1)

        pltpu.make_async_copy(ones0, degs.at[sblk.at[_chunk_rows(a) + 1]],
                              semA0).wait()

        @pl.when(cnt0 > 0.0)
        def _():
            pltpu.make_async_copy(ewf0, loops.at[lidx0], semB0).wait()

        pltpu.make_async_copy(ones1, degs.at[sblk.at[_chunk_rows(b) + 1]],
                              semA1).wait()

        @pl.when(cnt1 > 0.0)
        def _():
            pltpu.make_async_copy(ewf1, loops.at[lidx1], semB1).wait()

        return 0

    lax.fori_loop(0, NCH // 2, step, 0)

    plsc.subcore_barrier()
    pltpu.sync_copy(degs.at[pl.ds(sid * NSL, NSL)],
                    deg_hbm.at[cid, pl.ds(sid * NSL, NSL)])
    pltpu.sync_copy(loops.at[pl.ds(sid * NSL, NSL)],
                    lw_hbm.at[cid, pl.ds(sid * NSL, NSL)])


def _deg_loopw_partials(edges):
    mesh = plsc.VectorSubcoreMesh(core_axis_name="c", subcore_axis_name="s")
    k = pl.kernel(
        _deg_body,
        out_type=(
            jax.ShapeDtypeStruct((2, NP), jnp.float32),
            jax.ShapeDtypeStruct((2, NP), jnp.float32),
        ),
        mesh=mesh,
        scratch_types=[
            pltpu.VMEM((NSL,), jnp.float32),
            pltpu.VMEM((96, KC), jnp.int32),
            pltpu.VMEM((KC,), jnp.float32),
            pltpu.VMEM((KC,), jnp.float32),
            pltpu.VMEM((KC,), jnp.int32),
            pltpu.VMEM((KC,), jnp.int32),
            pltpu.VMEM((KC,), jnp.float32),
            pltpu.VMEM((KC,), jnp.float32),
            pltpu.VMEM_SHARED((NA,), jnp.float32),
            pltpu.VMEM_SHARED((NA,), jnp.float32),
            pltpu.SemaphoreType.DMA,
            pltpu.SemaphoreType.DMA,
            pltpu.SemaphoreType.DMA,
            pltpu.SemaphoreType.DMA,
        ],
    )
    return k(edges)


def _msg_body(edges_hbm, yp_hbm, out_hbm, sblk, buf0, buf1, acc, rowsem0,
              rowsem1, scatsem0, scatsem1):
    cid = lax.axis_index("c")
    sid = lax.axis_index("s")
    wid = cid * 16 + sid

    def zrow(i, _):
        for j in range(D // 16):
            buf0[i, pl.ds(j * 16, 16)] = jnp.zeros((16,), jnp.float32)
        return 0

    lax.fori_loop(0, KC, zrow, 0)
    for t in range(NSL // KC):
        pltpu.sync_copy(buf0, acc.at[pl.ds(sid * NSL + t * KC, KC)])

    @pl.when(wid == 0)
    def _():
        pltpu.sync_copy(buf0.at[pl.ds(0, 8)], acc.at[pl.ds(NP, 8)])

    zero = jnp.zeros((16,), jnp.float32)

    def scale(ch, buf):
        base = _chunk_rows(ch)

        def sgroup(g, _):
            r = sblk[base, pl.ds(g * 16, 16)]
            c = sblk[base + 1, pl.ds(g * 16, 16)]
            q = sblk[base + 2, pl.ds(g * 16, 16)]
            w = jnp.where(r == c, zero, q.astype(jnp.float32) * QS)
            for lane in range(16):
                s = w[lane]
                e = g * 16 + lane
                for j in range(D // 16):
                    buf[e, pl.ds(j * 16, 16)] = buf[e, pl.ds(j * 16, 16)] * s
            return 0

        lax.fori_loop(0, KC // 16, sgroup, 0)

    def gather(ch, buf, sem):
        pltpu.async_copy(yp_hbm.at[sblk.at[_chunk_rows(ch)]], buf, sem)

    def gather_wait(ch, buf, sem):
        pltpu.make_async_copy(yp_hbm.at[sblk.at[_chunk_rows(ch)]], buf,
                              sem).wait()

    def scat(ch, buf, sem):
        pltpu.async_copy(buf, acc.at[sblk.at[_chunk_rows(ch) + 1]], sem,
                         add=True)

    def scat_wait(ch, buf, sem):
        pltpu.make_async_copy(buf, acc.at[sblk.at[_chunk_rows(ch) + 1]],
                              sem).wait()

    _sb_load(edges_hbm, sblk, wid, 0)
    gather(0, buf0, rowsem0)
    gather(1, buf1, rowsem1)
    plsc.subcore_barrier()

    def step(i, _):
        @pl.when(jnp.logical_and(lax.bitwise_and(i, 7) == 0,
                                 lax.shift_right_logical(i, 3) + 1 < NSB))
        def _():
            _sb_load(edges_hbm, sblk, wid, lax.shift_right_logical(i, 3) + 1)

        a = 2 * i
        b = 2 * i + 1
        gather_wait(a, buf0, rowsem0)
        scale(a, buf0)
        scat(a, buf0, scatsem0)
        gather_wait(b, buf1, rowsem1)
        scale(b, buf1)
        scat(b, buf1, scatsem1)
        scat_wait(a, buf0, scatsem0)

        @pl.when(a + 2 < NCH)
        def _():
            gather(a + 2, buf0, rowsem0)

        scat_wait(b, buf1, scatsem1)

        @pl.when(b + 2 < NCH)
        def _():
            gather(b + 2, buf1, rowsem1)

        return 0

    lax.fori_loop(0, NCH // 2, step, 0)

    plsc.subcore_barrier()
    for t in range(NSL // KC):
        off = sid * NSL + t * KC
        pltpu.sync_copy(acc.at[pl.ds(off, KC)], out_hbm.at[cid, pl.ds(off, KC)])


def _msg_partials(edges, yp):
    mesh = plsc.VectorSubcoreMesh(core_axis_name="c", subcore_axis_name="s")
    k = pl.kernel(
        _msg_body,
        out_type=jax.ShapeDtypeStruct((2, NP, D), jnp.float32),
        mesh=mesh,
        scratch_types=[
            pltpu.VMEM((96, KC), jnp.int32),
            pltpu.VMEM((KC, D), jnp.float32),
            pltpu.VMEM((KC, D), jnp.float32),
            pltpu.VMEM_SHARED((NA, D), jnp.float32),
            pltpu.SemaphoreType.DMA,
            pltpu.SemaphoreType.DMA,
            pltpu.SemaphoreType.DMA,
            pltpu.SemaphoreType.DMA,
        ],
    )
    return k(edges, yp)


def _first_body(degp_ref, lwp_ref, x_ref, w_ref, b_ref, yp_ref, dis_ref,
                lw_ref):
    deg = jnp.sum(degp_ref[...], axis=0) + 1.0
    dis = lax.rsqrt(deg)[:, None]
    lw0 = lwp_ref[0]
    lw1 = lwp_ref[1]
    lw_ref[...] = jnp.where(lw0 == 1.0, lw1, lw0)[:, None]
    y = jnp.dot(x_ref[...], w_ref[...], preferred_element_type=jnp.float32)
    yp_ref[...] = (y + b_ref[...]) * dis
    dis_ref[...] = dis


def _first_stage(degp, lwp, x, wt, b):
    return pl.pallas_call(
        _first_body,
        grid=(NP // BR,),
        in_specs=[
            pl.BlockSpec((2, BR), lambda i: (0, i)),
            pl.BlockSpec((2, BR), lambda i: (0, i)),
            pl.BlockSpec((BR, D), lambda i: (i, 0)),
            pl.BlockSpec((D, D), lambda i: (0, 0)),
            pl.BlockSpec((1, D), lambda i: (0, 0)),
        ],
        out_specs=[
            pl.BlockSpec((BR, D), lambda i: (i, 0)),
            pl.BlockSpec((BR, 1), lambda i: (i, 0)),
            pl.BlockSpec((BR, 1), lambda i: (i, 0)),
        ],
        out_shape=[
            jax.ShapeDtypeStruct((NP, D), jnp.float32),
            jax.ShapeDtypeStruct((NP, 1), jnp.float32),
            jax.ShapeDtypeStruct((NP, 1), jnp.float32),
        ],
    )(degp, lwp, x, wt, b)


def _mid_body(sp_ref, yp_ref, lw_ref, dis_ref, w_ref, b_ref, out_ref):
    s = sp_ref[0] + sp_ref[1] + lw_ref[...] * yp_ref[...]
    h = jnp.maximum(s * dis_ref[...], 0.0)
    y = jnp.dot(h, w_ref[...], preferred_element_type=jnp.float32)
    out_ref[...] = (y + b_ref[...]) * dis_ref[...]


def _mid_stage(sp, yp, lw, dis, wt, b):
    return pl.pallas_call(
        _mid_body,
        grid=(NP // BR,),
        in_specs=[
            pl.BlockSpec((2, BR, D), lambda i: (0, i, 0)),
            pl.BlockSpec((BR, D), lambda i: (i, 0)),
            pl.BlockSpec((BR, 1), lambda i: (i, 0)),
            pl.BlockSpec((BR, 1), lambda i: (i, 0)),
            pl.BlockSpec((D, D), lambda i: (0, 0)),
            pl.BlockSpec((1, D), lambda i: (0, 0)),
        ],
        out_specs=pl.BlockSpec((BR, D), lambda i: (i, 0)),
        out_shape=jax.ShapeDtypeStruct((NP, D), jnp.float32),
    )(sp, yp, lw, dis, wt, b)


def _final_body(sp_ref, yp_ref, lw_ref, dis_ref, out_ref):
    s = sp_ref[0] + sp_ref[1] + lw_ref[...] * yp_ref[...]
    out_ref[...] = jax.nn.sigmoid(s * dis_ref[...])


def _final_stage(sp, yp, lw, dis):
    return pl.pallas_call(
        _final_body,
        grid=(NP // BR,),
        in_specs=[
            pl.BlockSpec((2, BR, D), lambda i: (0, i, 0)),
            pl.BlockSpec((BR, D), lambda i: (i, 0)),
            pl.BlockSpec((BR, 1), lambda i: (i, 0)),
            pl.BlockSpec((BR, 1), lambda i: (i, 0)),
        ],
        out_specs=pl.BlockSpec((BR, D), lambda i: (i, 0)),
        out_shape=jax.ShapeDtypeStruct((NP, D), jnp.float32),
    )(sp, yp, lw, dis)


@jax.jit
def kernel(x, edge_index, edge_weight, W0, b0, W1, b1):
    pad = NCH * KC - EPW
    rw = jnp.pad(edge_index[0].reshape(NW, EPW), ((0, 0), (0, pad)))
    cw = jnp.pad(edge_index[1].reshape(NW, EPW), ((0, 0), (0, pad)),
                 constant_values=NP)
    qw = jnp.pad(
        (edge_weight[:, 0] * 8388608.0 + 0.5).astype(jnp.int32).reshape(
            NW, EPW), ((0, 0), (0, pad)))
    edges = jnp.stack(
        [rw.reshape(NW, NCH, KC), cw.reshape(NW, NCH, KC),
         qw.reshape(NW, NCH, KC)], axis=2).reshape(NW, NSB, 48, KC)

    xp = jnp.zeros((NP, D), jnp.float32).at[:N].set(x)

    degp, lwp = _deg_loopw_partials(edges)
    yp0, dis, lw = _first_stage(degp, lwp, xp, W0.T, b0[None, :])
    sp0 = _msg_partials(edges, yp0)
    yp1 = _mid_stage(sp0, yp0, lw, dis, W1.T, b1[None, :])
    sp1 = _msg_partials(edges, yp1)
    out = _final_stage(sp1, yp1, lw, dis)
    return out[:N]


# speedup vs baseline: 11.2272x; 1.0000x over previous
import jax
import jax.numpy as jnp
from jax import lax
from jax.experimental import pallas as pl
from jax.experimental.pallas import tpu as pltpu
from jax.experimental.pallas import tpu_sc as plsc

N = 10000
D = 128
E = 320000
NP = 10240
NA = NP + 8
NW = 32
EPW = E // NW
KC = 128
NCH = 80
NSB = 5
BR = 640
NSL = NP // 16
QS = 1.0 / 8388608.0


def _chunk_rows(ch):
    return 3 * lax.bitwise_and(ch, 31)


def _sb_load(edges_hbm, sblk, wid, f):
    half = lax.bitwise_and(f, 1)
    pltpu.sync_copy(edges_hbm.at[wid, f], sblk.at[pl.ds(half * 48, 48)])


def _deg_body(edges_hbm, deg_hbm, lw_hbm, zbuf, sblk, ones0, ones1, lidx0,
              lidx1, ewf0, ewf1, degs, loops, semA0, semA1, semB0, semB1):
    cid = lax.axis_index("c")
    sid = lax.axis_index("s")
    wid = cid * 16 + sid

    def zbody(i, _):
        zbuf[pl.ds(i * 16, 16)] = jnp.zeros((16,), jnp.float32)
        return 0

    lax.fori_loop(0, NSL // 16, zbody, 0)
    pltpu.sync_copy(zbuf, degs.at[pl.ds(sid * NSL, NSL)])

    def obody(i, _):
        zbuf[pl.ds(i * 16, 16)] = jnp.full((16,), 1.0, jnp.float32)
        return 0

    lax.fori_loop(0, NSL // 16, obody, 0)
    pltpu.sync_copy(zbuf, loops.at[pl.ds(sid * NSL, NSL)])

    @pl.when(sid == 0)
    def _():
        pltpu.sync_copy(zbuf.at[pl.ds(0, 8)], degs.at[pl.ds(NP, 8)])
        pltpu.sync_copy(zbuf.at[pl.ds(0, 8)], loops.at[pl.ds(NP, 8)])

    plsc.subcore_barrier()

    one = jnp.full((16,), 1.0, jnp.float32)
    zero = jnp.zeros((16,), jnp.float32)
    trash = jnp.full((16,), NP, jnp.int32)

    _sb_load(edges_hbm, sblk, wid, 0)

    def compute(ch, onesv, lidxv, ewf):
        base = _chunk_rows(ch)

        def group(g, cntv):
            r = sblk[base, pl.ds(g * 16, 16)]
            c = sblk[base + 1, pl.ds(g * 16, 16)]
            q = sblk[base + 2, pl.ds(g * 16, 16)]
            is_loop = r == c
            ones = jnp.where(is_loop, zero, one)
            onesv[pl.ds(g * 16, 16)] = ones
            lidxv[pl.ds(g * 16, 16)] = jnp.where(is_loop, c, trash)
            ewf[pl.ds(g * 16, 16)] = q.astype(jnp.float32) * QS
            return cntv + (one - ones)

        cntv = lax.fori_loop(0, KC // 16, group, zero)
        cnt = cntv[0]
        for k in range(1, 16):
            cnt = cnt + cntv[k]
        return cnt

    def step(i, _):
        @pl.when(jnp.logical_and(lax.bitwise_and(i, 7) == 0,
                                 lax.shift_right_logical(i, 3) + 1 < NSB))
        def _():
            _sb_load(edges_hbm, sblk, wid, lax.shift_right_logical(i, 3) + 1)

        a = 2 * i
        b = 2 * i + 1
        cnt0 = compute(a, ones0, lidx0, ewf0)
        pltpu.async_copy(ones0, degs.at[sblk.at[_chunk_rows(a) + 1]], semA0,
                         add=True)

        @pl.when(cnt0 > 0.0)
        def _():
            pltpu.async_copy(ewf0, loops.at[lidx0], semB0)

        cnt1 = compute(b, ones1, lidx1, ewf1)
        pltpu.async_copy(ones1, degs.at[sblk.at[_chunk_rows(b) + 1]], semA1,
                         add=True)

        @pl.when(cnt1 > 0.0)
        def _():
            pltpu.async_copy(ewf1, loops.at[lidx1], semB1)

        pltpu.make_async_copy(ones0, degs.at[sblk.at[_chunk_rows(a) + 1]],
                              semA0).wait()

        @pl.when(cnt0 > 0.0)
        def _():
            pltpu.make_async_copy(ewf0, loops.at[lidx0], semB0).wait()

        pltpu.make_async_copy(ones1, degs.at[sblk.at[_chunk_rows(b) + 1]],
                              semA1).wait()

        @pl.when(cnt1 > 0.0)
        def _():
            pltpu.make_async_copy(ewf1, loops.at[lidx1], semB1).wait()

        return 0

    lax.fori_loop(0, NCH // 2, step, 0)

    plsc.subcore_barrier()
    pltpu.sync_copy(degs.at[pl.ds(sid * NSL, NSL)],
                    deg_hbm.at[cid, pl.ds(sid * NSL, NSL)])
    pltpu.sync_copy(loops.at[pl.ds(sid * NSL, NSL)],
                    lw_hbm.at[cid, pl.ds(sid * NSL, NSL)])


def _deg_loopw_partials(edges):
    mesh = plsc.VectorSubcoreMesh(core_axis_name="c", subcore_axis_name="s")
    k = pl.kernel(
        _deg_body,
        out_type=(
            jax.ShapeDtypeStruct((2, NP), jnp.float32),
            jax.ShapeDtypeStruct((2, NP), jnp.float32),
        ),
        mesh=mesh,
        scratch_types=[
            pltpu.VMEM((NSL,), jnp.float32),
            pltpu.VMEM((96, KC), jnp.int32),
            pltpu.VMEM((KC,), jnp.float32),
            pltpu.VMEM((KC,), jnp.float32),
            pltpu.VMEM((KC,), jnp.int32),
            pltpu.VMEM((KC,), jnp.int32),
            pltpu.VMEM((KC,), jnp.float32),
            pltpu.VMEM((KC,), jnp.float32),
            pltpu.VMEM_SHARED((NA,), jnp.float32),
            pltpu.VMEM_SHARED((NA,), jnp.float32),
            pltpu.SemaphoreType.DMA,
            pltpu.SemaphoreType.DMA,
            pltpu.SemaphoreType.DMA,
            pltpu.SemaphoreType.DMA,
        ],
    )
    return k(edges)


def _msg_body(edges_hbm, yp_hbm, out_hbm, idx0, idx1, colv0, colv1, buf0,
              buf1, acc, rowsem0, rowsem1, scatsem0, scatsem1, ldsem0,
              ldsem1):
    cid = lax.axis_index("c")
    sid = lax.axis_index("s")
    wid = cid * 16 + sid

    def zrow(i, _):
        for j in range(D // 16):
            buf0[i, pl.ds(j * 16, 16)] = jnp.zeros((16,), jnp.float32)
        return 0

    lax.fori_loop(0, KC, zrow, 0)
    for t in range(NSL // KC):
        pltpu.sync_copy(buf0, acc.at[pl.ds(sid * NSL + t * KC, KC)])

    @pl.when(wid == 0)
    def _():
        pltpu.sync_copy(buf0.at[pl.ds(0, 8)], acc.at[pl.ds(NP, 8)])

    zero = jnp.zeros((16,), jnp.float32)

    def scale(idxb, colv, buf):
        def sgroup(g, _):
            r = idxb[0, pl.ds(g * 16, 16)]
            c = idxb[1, pl.ds(g * 16, 16)]
            q = idxb[2, pl.ds(g * 16, 16)]
            colv[pl.ds(g * 16, 16)] = c
            w = jnp.where(r == c, zero, q.astype(jnp.float32) * QS)
            for lane in range(16):
                s = w[lane]
                e = g * 16 + lane
                for j in range(D // 16):
                    buf[e, pl.ds(j * 16, 16)] = buf[e, pl.ds(j * 16, 16)] * s
            return 0

        lax.fori_loop(0, KC // 16, sgroup, 0)

    pltpu.sync_copy(edges_hbm.at[wid, 0], idx0)
    pltpu.async_copy(yp_hbm.at[idx0.at[0]], buf0, rowsem0)
    pltpu.sync_copy(edges_hbm.at[wid, 1], idx1)
    pltpu.async_copy(yp_hbm.at[idx1.at[0]], buf1, rowsem1)
    plsc.subcore_barrier()

    def step(i, _):
        a = 2 * i
        b = 2 * i + 1
        pltpu.make_async_copy(yp_hbm.at[idx0.at[0]], buf0, rowsem0).wait()
        scale(idx0, colv0, buf0)
        pltpu.async_copy(buf0, acc.at[colv0], scatsem0, add=True)

        @pl.when(a + 2 < NCH)
        def _():
            pltpu.async_copy(edges_hbm.at[wid, a + 2], idx0, ldsem0)

        pltpu.make_async_copy(yp_hbm.at[idx1.at[0]], buf1, rowsem1).wait()
        scale(idx1, colv1, buf1)
        pltpu.async_copy(buf1, acc.at[colv1], scatsem1, add=True)

        @pl.when(b + 2 < NCH)
        def _():
            pltpu.async_copy(edges_hbm.at[wid, b + 2], idx1, ldsem1)

        pltpu.make_async_copy(buf0, acc.at[colv0], scatsem0).wait()

        @pl.when(a + 2 < NCH)
        def _():
            pltpu.make_async_copy(edges_hbm.at[wid, a + 2], idx0, ldsem0).wait()
            pltpu.async_copy(yp_hbm.at[idx0.at[0]], buf0, rowsem0)

        pltpu.make_async_copy(buf1, acc.at[colv1], scatsem1).wait()

        @pl.when(b + 2 < NCH)
        def _():
            pltpu.make_async_copy(edges_hbm.at[wid, b + 2], idx1, ldsem1).wait()
            pltpu.async_copy(yp_hbm.at[idx1.at[0]], buf1, rowsem1)

        return 0

    lax.fori_loop(0, NCH // 2, step, 0)

    plsc.subcore_barrier()
    for t in range(NSL // KC):
        off = sid * NSL + t * KC
        pltpu.sync_copy(acc.at[pl.ds(off, KC)], out_hbm.at[cid, pl.ds(off, KC)])


def _msg_partials(edges, yp):
    mesh = plsc.VectorSubcoreMesh(core_axis_name="c", subcore_axis_name="s")
    k = pl.kernel(
        _msg_body,
        out_type=jax.ShapeDtypeStruct((2, NP, D), jnp.float32),
        mesh=mesh,
        scratch_types=[
            pltpu.VMEM((3, KC), jnp.int32),
            pltpu.VMEM((3, KC), jnp.int32),
            pltpu.VMEM((KC,), jnp.int32),
            pltpu.VMEM((KC,), jnp.int32),
            pltpu.VMEM((KC, D), jnp.float32),
            pltpu.VMEM((KC, D), jnp.float32),
            pltpu.VMEM_SHARED((NA, D), jnp.float32),
            pltpu.SemaphoreType.DMA,
            pltpu.SemaphoreType.DMA,
            pltpu.SemaphoreType.DMA,
            pltpu.SemaphoreType.DMA,
            pltpu.SemaphoreType.DMA,
            pltpu.SemaphoreType.DMA,
        ],
    )
    return k(edges, yp)


def _first_body(degp_ref, lwp_ref, x_ref, w_ref, b_ref, yp_ref, dis_ref,
                lw_ref):
    deg = jnp.sum(degp_ref[...], axis=0) + 1.0
    dis = lax.rsqrt(deg)[:, None]
    lw0 = lwp_ref[0]
    lw1 = lwp_ref[1]
    lw_ref[...] = jnp.where(lw0 == 1.0, lw1, lw0)[:, None]
    y = jnp.dot(x_ref[...], w_ref[...], preferred_element_type=jnp.float32)
    yp_ref[...] = (y + b_ref[...]) * dis
    dis_ref[...] = dis


def _first_stage(degp, lwp, x, wt, b):
    return pl.pallas_call(
        _first_body,
        grid=(NP // BR,),
        in_specs=[
            pl.BlockSpec((2, BR), lambda i: (0, i)),
            pl.BlockSpec((2, BR), lambda i: (0, i)),
            pl.BlockSpec((BR, D), lambda i: (i, 0)),
            pl.BlockSpec((D, D), lambda i: (0, 0)),
            pl.BlockSpec((1, D), lambda i: (0, 0)),
        ],
        out_specs=[
            pl.BlockSpec((BR, D), lambda i: (i, 0)),
            pl.BlockSpec((BR, 1), lambda i: (i, 0)),
            pl.BlockSpec((BR, 1), lambda i: (i, 0)),
        ],
        out_shape=[
            jax.ShapeDtypeStruct((NP, D), jnp.float32),
            jax.ShapeDtypeStruct((NP, 1), jnp.float32),
            jax.ShapeDtypeStruct((NP, 1), jnp.float32),
        ],
    )(degp, lwp, x, wt, b)


def _mid_body(sp_ref, yp_ref, lw_ref, dis_ref, w_ref, b_ref, out_ref):
    s = sp_ref[0] + sp_ref[1] + lw_ref[...] * yp_ref[...]
    h = jnp.maximum(s * dis_ref[...], 0.0)
    y = jnp.dot(h, w_ref[...], preferred_element_type=jnp.float32)
    out_ref[...] = (y + b_ref[...]) * dis_ref[...]


def _mid_stage(sp, yp, lw, dis, wt, b):
    return pl.pallas_call(
        _mid_body,
        grid=(NP // BR,),
        in_specs=[
            pl.BlockSpec((2, BR, D), lambda i: (0, i, 0)),
            pl.BlockSpec((BR, D), lambda i: (i, 0)),
            pl.BlockSpec((BR, 1), lambda i: (i, 0)),
            pl.BlockSpec((BR, 1), lambda i: (i, 0)),
            pl.BlockSpec((D, D), lambda i: (0, 0)),
            pl.BlockSpec((1, D), lambda i: (0, 0)),
        ],
        out_specs=pl.BlockSpec((BR, D), lambda i: (i, 0)),
        out_shape=jax.ShapeDtypeStruct((NP, D), jnp.float32),
    )(sp, yp, lw, dis, wt, b)


def _final_body(sp_ref, yp_ref, lw_ref, dis_ref, out_ref):
    s = sp_ref[0] + sp_ref[1] + lw_ref[...] * yp_ref[...]
    out_ref[...] = jax.nn.sigmoid(s * dis_ref[...])


def _final_stage(sp, yp, lw, dis):
    return pl.pallas_call(
        _final_body,
        grid=(NP // BR,),
        in_specs=[
            pl.BlockSpec((2, BR, D), lambda i: (0, i, 0)),
            pl.BlockSpec((BR, D), lambda i: (i, 0)),
            pl.BlockSpec((BR, 1), lambda i: (i, 0)),
            pl.BlockSpec((BR, 1), lambda i: (i, 0)),
        ],
        out_specs=pl.BlockSpec((BR, D), lambda i: (i, 0)),
        out_shape=jax.ShapeDtypeStruct((NP, D), jnp.float32),
    )(sp, yp, lw, dis)


@jax.jit
def kernel(x, edge_index, edge_weight, W0, b0, W1, b1):
    pad = NCH * KC - EPW
    rw = jnp.pad(edge_index[0].reshape(NW, EPW), ((0, 0), (0, pad)))
    cw = jnp.pad(edge_index[1].reshape(NW, EPW), ((0, 0), (0, pad)),
                 constant_values=NP)
    qw = jnp.pad(
        (edge_weight[:, 0] * 8388608.0 + 0.5).astype(jnp.int32).reshape(
            NW, EPW), ((0, 0), (0, pad)))
    edges = jnp.stack(
        [rw.reshape(NW, NCH, KC), cw.reshape(NW, NCH, KC),
         qw.reshape(NW, NCH, KC)], axis=2)
    edges_sb = edges.reshape(NW, NSB, 48, KC)

    xp = jnp.zeros((NP, D), jnp.float32).at[:N].set(x)

    degp, lwp = _deg_loopw_partials(edges_sb)
    yp0, dis, lw = _first_stage(degp, lwp, xp, W0.T, b0[None, :])
    sp0 = _msg_partials(edges, yp0)
    yp1 = _mid_stage(sp0, yp0, lw, dis, W1.T, b1[None, :])
    sp1 = _msg_partials(edges, yp1)
    out = _final_stage(sp1, yp1, lw, dis)
    return out[:N]
